# baseline (speedup 1.0000x reference)
import functools

import jax
import jax.numpy as jnp
from jax import lax
from jax.experimental import pallas as pl
from jax.experimental.pallas import tpu as pltpu
from jax.experimental.pallas import tpu_sc as plsc

N = 2048
D = 768
F = 3072
E = 8
K = 2
CAP = 512
FT = 1024
NF = F // FT
RC = 256
BIG = 1 << 20
NW = 32
BPW = E * CAP // NW


def _iscan(x):
    p = x
    sh = 1
    while sh < N:
        p = p + jnp.concatenate(
            [jnp.zeros((1, sh), x.dtype), p[:, :-sh]], axis=1)
        sh *= 2
    return p


def _dispatch_kernel(logits_ref, ti_ref, slot_ref, w_ref, cand_ref,
                     lgT_s, pT_s):
    e = pl.program_id(0)
    i32 = jnp.int32
    u32 = jnp.uint32

    @pl.when(e == 0)
    def _init():
        lgT = jnp.swapaxes(logits_ref[:], 0, 1)
        lgT_s[:] = lgT
        pT_s[:] = jax.nn.softmax(lgT, axis=0)

    lgT = lgT_s[:]
    iota8 = jax.lax.broadcasted_iota(i32, (E, N), 0)
    big = jnp.int32(1 << 30)
    m1 = jnp.max(lgT, axis=0, keepdims=True)
    i1r = jnp.min(jnp.where(lgT == m1, iota8, big), axis=0, keepdims=True)
    lg2 = jnp.where(iota8 == i1r, -jnp.inf, lgT)
    m2 = jnp.max(lg2, axis=0, keepdims=True)
    i2r = jnp.min(jnp.where(lg2 == m2, iota8, big), axis=0, keepdims=True)

    @pl.when(e == 0)
    def _write_ti():
        ti_ref[:] = jnp.swapaxes(
            jnp.concatenate([i1r, i2r], axis=0), 0, 1)

    routed = (i1r == e) | (i2r == e)
    krow = jnp.max(jnp.where(iota8 == e, lgT, -jnp.inf),
                   axis=0, keepdims=True)
    prow = jnp.sum(jnp.where(iota8 == e, pT_s[:], 0.0),
                   axis=0, keepdims=True)
    key = jnp.where(routed, krow, -jnp.inf) + 0.0

    bu = jax.lax.bitcast_convert_type(key, u32)
    u = jnp.where((bu >> 31) == 0, bu | jnp.uint32(0x80000000), ~bu)
    u = jnp.where(routed, u, jnp.uint32(0))

    def body(_, carry):
        lo, hi = carry
        mid = lo + ((hi - lo + jnp.uint32(1)) >> 1)
        cnt = jnp.sum((u >= mid).astype(i32))
        ok = cnt >= CAP
        return (jnp.where(ok, mid, lo), jnp.where(ok, hi, mid - 1))

    lo, _ = jax.lax.fori_loop(
        0, 32, body, (jnp.uint32(0), jnp.uint32(0xFFFFFFFE)))

    gt = routed & (u > lo)
    need = CAP - jnp.sum(gt.astype(i32))
    tie = routed & (u == lo)
    tp = _iscan(tie.astype(i32)) - tie.astype(i32)
    keep = gt | (tie & (tp < need))
    ki = keep.astype(i32)
    slot = _iscan(ki) - ki
    slot_row = jnp.where(keep, slot, BIG)
    slot_ref[:] = slot_row.reshape(1, 1, N)
    w_ref[:] = prow.reshape(1, 1, N)

    iota_s = jax.lax.broadcasted_iota(i32, (CAP, 1), 0)
    oh = jnp.where(iota_s == slot_row, 1.0, 0.0)
    iota_t = jax.lax.broadcasted_iota(i32, (1, N), 1).astype(jnp.float32)
    cand_ref[:] = jnp.sum(oh * iota_t, axis=1, keepdims=True).astype(
        i32).reshape(1, CAP, 1)


def _sc_gather(xf, cand):
    mesh = plsc.VectorSubcoreMesh(core_axis_name="c", subcore_axis_name="s")

    @functools.partial(
        pl.kernel, mesh=mesh,
        out_type=jax.ShapeDtypeStruct((E * CAP, D), jnp.float32),
        scratch_types=[
            pltpu.VMEM((BPW,), jnp.int32),
            pltpu.VMEM((BPW, D), jnp.float32),
            pltpu.SemaphoreType.DMA,
        ],
    )
    def k(xf_hbm, cand_hbm, out_hbm, idx_v, rows_v, sem):
        wid = lax.axis_index("s") * 2 + lax.axis_index("c")
        base = wid * BPW
        pltpu.sync_copy(cand_hbm.at[pl.ds(base, BPW)], idx_v)
        pltpu.async_copy(xf_hbm.at[idx_v], rows_v, sem).wait()
        pltpu.sync_copy(rows_v, out_hbm.at[pl.ds(base, BPW)])

    return k(xf, cand)


def _ffn_kernel(rank_ref, w_ref, xe_ref, gw_ref, gb_ref, vw_ref, vb_ref,
                ww_ref, wb_ref, out_ref, onehot_s, xe_s, oe_s, w_s):
    e = pl.program_id(0)
    f = pl.program_id(1)
    f32 = jnp.float32

    @pl.when(jnp.logical_and(e == 0, f == 0))
    def _init():
        out_ref[:] = jnp.zeros_like(out_ref)

    bf16 = jnp.bfloat16

    @pl.when(f == 0)
    def _gather():
        rank_row = rank_ref[0]
        iota_s = jax.lax.broadcasted_iota(jnp.int32, (CAP, 1), 0)
        oh = jnp.where(iota_s == rank_row, 1.0, 0.0).astype(bf16)
        onehot_s[:] = oh
        w_s[:] = jnp.sum(oh.astype(f32) * w_ref[0], axis=1, keepdims=True)
        xe_s[:] = xe_ref[:].astype(bf16)
        oe_s[:] = jnp.broadcast_to(wb_ref[0], (CAP, D))

    xe = xe_s[:]
    g = jnp.dot(xe, gw_ref[0].astype(bf16),
                preferred_element_type=f32) + gb_ref[0]
    v = jnp.dot(xe, vw_ref[0].astype(bf16),
                preferred_element_type=f32) + vb_ref[0]
    h = v * (g * jax.nn.sigmoid(g))
    oe_s[:] += jnp.dot(h.astype(bf16), ww_ref[0].astype(bf16),
                       preferred_element_type=f32)

    @pl.when(f == NF - 1)
    def _combine():
        contrib = oe_s[:] * w_s[:]
        out_ref[:] += jax.lax.dot_general(
            onehot_s[:], contrib.astype(bf16),
            dimension_numbers=(((0,), (0,)), ((), ())),
            preferred_element_type=f32)


@functools.partial(jax.jit, static_argnames=("interpret",))
def _moe(logits, xf, gate_W, gate_b, val_W, val_b, wo_W, wo_b,
         interpret=False):
    ti, rank, w, cand = pl.pallas_call(
        _dispatch_kernel,
        grid=(E,),
        in_specs=[pl.BlockSpec((N, E), lambda e: (0, 0))],
        out_specs=[
            pl.BlockSpec((N, K), lambda e: (0, 0)),
            pl.BlockSpec((1, 1, N), lambda e: (e, 0, 0)),
            pl.BlockSpec((1, 1, N), lambda e: (e, 0, 0)),
            pl.BlockSpec((1, CAP, 1), lambda e: (e, 0, 0)),
        ],
        out_shape=[
            jax.ShapeDtypeStruct((N, K), jnp.int32),
            jax.ShapeDtypeStruct((E, 1, N), jnp.int32),
            jax.ShapeDtypeStruct((E, 1, N), jnp.float32),
            jax.ShapeDtypeStruct((E, CAP, 1), jnp.int32),
        ],
        scratch_shapes=[pltpu.VMEM((E, N), jnp.float32),
                        pltpu.VMEM((E, N), jnp.float32)],
        interpret=interpret,
    )(logits)

    xe = _sc_gather(xf, cand.reshape(E * CAP))

    out = pl.pallas_call(
        _ffn_kernel,
        grid=(E, NF),
        in_specs=[
            pl.BlockSpec((1, 1, N), lambda e, f: (e, 0, 0)),
            pl.BlockSpec((1, 1, N), lambda e, f: (e, 0, 0)),
            pl.BlockSpec((CAP, D), lambda e, f: (e, 0)),
            pl.BlockSpec((1, D, FT), lambda e, f: (e, 0, f)),
            pl.BlockSpec((1, 1, FT), lambda e, f: (e, 0, f)),
            pl.BlockSpec((1, D, FT), lambda e, f: (e, 0, f)),
            pl.BlockSpec((1, 1, FT), lambda e, f: (e, 0, f)),
            pl.BlockSpec((1, FT, D), lambda e, f: (e, f, 0)),
            pl.BlockSpec((1, 1, D), lambda e, f: (e, 0, 0)),
        ],
        out_specs=pl.BlockSpec((N, D), lambda e, f: (0, 0)),
        out_shape=jax.ShapeDtypeStruct((N, D), jnp.float32),
        scratch_shapes=[
            pltpu.VMEM((CAP, N), jnp.bfloat16),
            pltpu.VMEM((CAP, D), jnp.bfloat16),
            pltpu.VMEM((CAP, D), jnp.float32),
            pltpu.VMEM((CAP, 1), jnp.float32),
        ],
        interpret=interpret,
    )(rank, w, xe, gate_W,
      gate_b.reshape(E, 1, F), val_W,
      val_b.reshape(E, 1, F), wo_W,
      wo_b.reshape(E, 1, D))
    return out, ti


def kernel(x, Wr, br, gate_W, gate_b, val_W, val_b, wo_W, wo_b):
    b, s, d = x.shape
    xf = x.reshape(b * s, d)
    logits = xf @ Wr + br
    out, ti = _moe(logits, xf, gate_W, gate_b, val_W, val_b, wo_W, wo_b)
    return out.reshape(b, s, d), (logits, ti)


# speedup vs baseline: 1.4141x; 1.4141x over previous
import functools

import jax
import jax.numpy as jnp
from jax.experimental import pallas as pl
from jax.experimental.pallas import tpu as pltpu

N = 2048
D = 768
F = 3072
E = 8
K = 2
CAP = 512
FT = 1024
NF = F // FT
RC = 256
BIG = 1 << 20


def _iscan(x):
    p = x
    r = x.shape[0]
    sh = 1
    while sh < N:
        p = p + jnp.concatenate(
            [jnp.zeros((r, sh), x.dtype), p[:, :-sh]], axis=1)
        sh *= 2
    return p


def _dispatch_kernel(logits_ref, ti_ref, slot_ref, w_ref):
    i32 = jnp.int32
    u32 = jnp.uint32

    lgT = jnp.swapaxes(logits_ref[:], 0, 1)
    pT = jax.nn.softmax(lgT, axis=0)
    iota8 = jax.lax.broadcasted_iota(i32, (E, N), 0)
    big = jnp.int32(1 << 30)
    m1 = jnp.max(lgT, axis=0, keepdims=True)
    i1r = jnp.min(jnp.where(lgT == m1, iota8, big), axis=0, keepdims=True)
    lg2 = jnp.where(iota8 == i1r, -jnp.inf, lgT)
    m2 = jnp.max(lg2, axis=0, keepdims=True)
    i2r = jnp.min(jnp.where(lg2 == m2, iota8, big), axis=0, keepdims=True)
    ti_ref[:] = jnp.swapaxes(
        jnp.concatenate([i1r, i2r], axis=0), 0, 1)

    routed = (i1r == iota8) | (i2r == iota8)
    key = jnp.where(routed, lgT, -jnp.inf) + 0.0

    bu = jax.lax.bitcast_convert_type(key, u32)
    u = jnp.where((bu >> 31) == 0, bu | jnp.uint32(0x80000000), ~bu)
    u = jnp.where(routed, u, jnp.uint32(0))

    def body(_, carry):
        lo, hi = carry
        mid = lo + ((hi - lo + jnp.uint32(1)) >> 1)
        cnt = jnp.sum((u >= mid).astype(i32), axis=1, keepdims=True)
        ok = cnt >= CAP
        return (jnp.where(ok, mid, lo), jnp.where(ok, hi, mid - 1))

    lo, _ = jax.lax.fori_loop(
        0, 32, body, (jnp.full((E, 1), 0, u32),
                      jnp.full((E, 1), 0xFFFFFFFE, u32)))

    gt = routed & (u > lo)
    need = CAP - jnp.sum(gt.astype(i32), axis=1, keepdims=True)
    tie = routed & (u == lo)
    tp = _iscan(tie.astype(i32)) - tie.astype(i32)
    keep = gt | (tie & (tp < need))
    ki = keep.astype(i32)
    slot = _iscan(ki) - ki
    slot_ref[:] = jnp.where(keep, slot, BIG).reshape(E, 1, N)
    w_ref[:] = pT.reshape(E, 1, N)


def _ffn_kernel(rank_ref, w_ref, xf_ref, gw_ref, gb_ref, vw_ref, vb_ref,
                ww_ref, wb_ref, out_ref, onehot_s, xe_s, oe_s, w_s):
    e = pl.program_id(0)
    f = pl.program_id(1)
    f32 = jnp.float32

    @pl.when(jnp.logical_and(e == 0, f == 0))
    def _init():
        out_ref[:] = jnp.zeros_like(out_ref)

    bf16 = jnp.bfloat16

    @pl.when(f == 0)
    def _gather():
        rank_row = rank_ref[0]
        iota_s = jax.lax.broadcasted_iota(jnp.int32, (CAP, 1), 0)
        oh = jnp.where(iota_s == rank_row, 1.0, 0.0).astype(bf16)
        onehot_s[:] = oh
        w_s[:] = jnp.sum(oh.astype(f32) * w_ref[0], axis=1, keepdims=True)
        xe_s[:] = jnp.dot(oh, xf_ref[:],
                          preferred_element_type=f32).astype(bf16)
        oe_s[:] = jnp.broadcast_to(wb_ref[0], (CAP, D))

    xe = xe_s[:]
    g = jnp.dot(xe, gw_ref[0].astype(bf16),
                preferred_element_type=f32) + gb_ref[0]
    v = jnp.dot(xe, vw_ref[0].astype(bf16),
                preferred_element_type=f32) + vb_ref[0]
    h = v * (g * jax.nn.sigmoid(g))
    oe_s[:] += jnp.dot(h.astype(bf16), ww_ref[0].astype(bf16),
                       preferred_element_type=f32)

    @pl.when(f == NF - 1)
    def _combine():
        contrib = oe_s[:] * w_s[:]
        out_ref[:] += jax.lax.dot_general(
            onehot_s[:], contrib.astype(bf16),
            dimension_numbers=(((0,), (0,)), ((), ())),
            preferred_element_type=f32)


@functools.partial(jax.jit, static_argnames=("interpret",))
def _moe(logits, xf, gate_W, gate_b, val_W, val_b, wo_W, wo_b,
         interpret=False):
    ti, rank, w = pl.pallas_call(
        _dispatch_kernel,
        out_shape=[
            jax.ShapeDtypeStruct((N, K), jnp.int32),
            jax.ShapeDtypeStruct((E, 1, N), jnp.int32),
            jax.ShapeDtypeStruct((E, 1, N), jnp.float32),
        ],
        interpret=interpret,
    )(logits)

    out = pl.pallas_call(
        _ffn_kernel,
        grid=(E, NF),
        in_specs=[
            pl.BlockSpec((1, 1, N), lambda e, f: (e, 0, 0)),
            pl.BlockSpec((1, 1, N), lambda e, f: (e, 0, 0)),
            pl.BlockSpec((N, D), lambda e, f: (0, 0)),
            pl.BlockSpec((1, D, FT), lambda e, f: (e, 0, f)),
            pl.BlockSpec((1, 1, FT), lambda e, f: (e, 0, f)),
            pl.BlockSpec((1, D, FT), lambda e, f: (e, 0, f)),
            pl.BlockSpec((1, 1, FT), lambda e, f: (e, 0, f)),
            pl.BlockSpec((1, FT, D), lambda e, f: (e, f, 0)),
            pl.BlockSpec((1, 1, D), lambda e, f: (e, 0, 0)),
        ],
        out_specs=pl.BlockSpec((N, D), lambda e, f: (0, 0)),
        out_shape=jax.ShapeDtypeStruct((N, D), jnp.float32),
        scratch_shapes=[
            pltpu.VMEM((CAP, N), jnp.bfloat16),
            pltpu.VMEM((CAP, D), jnp.bfloat16),
            pltpu.VMEM((CAP, D), jnp.float32),
            pltpu.VMEM((CAP, 1), jnp.float32),
        ],
        interpret=interpret,
    )(rank, w, xf.astype(jnp.bfloat16), gate_W,
      gate_b.reshape(E, 1, F), val_W,
      val_b.reshape(E, 1, F), wo_W,
      wo_b.reshape(E, 1, D))
    return out, ti


def kernel(x, Wr, br, gate_W, gate_b, val_W, val_b, wo_W, wo_b):
    b, s, d = x.shape
    xf = x.reshape(b * s, d)
    logits = xf @ Wr + br
    out, ti = _moe(logits, xf, gate_W, gate_b, val_W, val_b, wo_W, wo_b)
    return out.reshape(b, s, d), (logits, ti)


# speedup vs baseline: 1.4747x; 1.0428x over previous
import functools

import jax
import jax.numpy as jnp
from jax.experimental import pallas as pl
from jax.experimental.pallas import tpu as pltpu

N = 2048
D = 768
F = 3072
E = 8
K = 2
CAP = 512
FT = 1536
NF = F // FT
RC = 256
BIG = 1 << 20


def _iscan(x):
    p = x
    r = x.shape[0]
    sh = 1
    while sh < N:
        p = p + jnp.concatenate(
            [jnp.zeros((r, sh), x.dtype), p[:, :-sh]], axis=1)
        sh *= 2
    return p


def _dispatch_kernel(logits_ref, ti_ref, slot_ref, w_ref):
    i32 = jnp.int32
    u32 = jnp.uint32

    lgT = jnp.swapaxes(logits_ref[:], 0, 1)
    pT = jax.nn.softmax(lgT, axis=0)
    iota8 = jax.lax.broadcasted_iota(i32, (E, N), 0)
    big = jnp.int32(1 << 30)
    m1 = jnp.max(lgT, axis=0, keepdims=True)
    i1r = jnp.min(jnp.where(lgT == m1, iota8, big), axis=0, keepdims=True)
    lg2 = jnp.where(iota8 == i1r, -jnp.inf, lgT)
    m2 = jnp.max(lg2, axis=0, keepdims=True)
    i2r = jnp.min(jnp.where(lg2 == m2, iota8, big), axis=0, keepdims=True)
    ti_ref[:] = jnp.swapaxes(
        jnp.concatenate([i1r, i2r], axis=0), 0, 1)

    routed = (i1r == iota8) | (i2r == iota8)
    key = jnp.where(routed, lgT, -jnp.inf) + 0.0

    bu = jax.lax.bitcast_convert_type(key, u32)
    u = jnp.where((bu >> 31) == 0, bu | jnp.uint32(0x80000000), ~bu)
    u = jnp.where(routed, u, jnp.uint32(0))

    def body(_, carry):
        lo, hi = carry
        mid = lo + ((hi - lo + jnp.uint32(1)) >> 1)
        cnt = jnp.sum((u >= mid).astype(i32), axis=1, keepdims=True)
        ok = cnt >= CAP
        return (jnp.where(ok, mid, lo), jnp.where(ok, hi, mid - 1))

    lo, _ = jax.lax.fori_loop(
        0, 32, body, (jnp.full((E, 1), 0, u32),
                      jnp.full((E, 1), 0xFFFFFFFE, u32)))

    gt = routed & (u > lo)
    need = CAP - jnp.sum(gt.astype(i32), axis=1, keepdims=True)
    tie = routed & (u == lo)
    tp = _iscan(tie.astype(i32)) - tie.astype(i32)
    keep = gt | (tie & (tp < need))
    ki = keep.astype(i32)
    slot = _iscan(ki) - ki
    slot_ref[:] = jnp.where(keep, slot, BIG).reshape(E, 1, N)
    w_ref[:] = pT.reshape(E, 1, N)


def _ffn_kernel(rank_ref, w_ref, xf_ref, gw_ref, gb_ref, vw_ref, vb_ref,
                ww_ref, wb_ref, out_ref, onehot_s, xe_s, oe_s, w_s):
    e = pl.program_id(0)
    f = pl.program_id(1)
    f32 = jnp.float32

    @pl.when(jnp.logical_and(e == 0, f == 0))
    def _init():
        out_ref[:] = jnp.zeros_like(out_ref)

    bf16 = jnp.bfloat16

    @pl.when(f == 0)
    def _gather():
        rank_row = rank_ref[0]
        iota_s = jax.lax.broadcasted_iota(jnp.int32, (CAP, 1), 0)
        oh = jnp.where(iota_s == rank_row, 1.0, 0.0).astype(bf16)
        onehot_s[:] = oh
        w_s[:] = jnp.sum(oh.astype(f32) * w_ref[0], axis=1, keepdims=True)
        xe_s[:] = jnp.dot(oh, xf_ref[:],
                          preferred_element_type=f32).astype(bf16)
        oe_s[:] = jnp.broadcast_to(wb_ref[0], (CAP, D))

    xe = xe_s[:]
    g = jnp.dot(xe, gw_ref[0].astype(bf16),
                preferred_element_type=f32) + gb_ref[0]
    v = jnp.dot(xe, vw_ref[0].astype(bf16),
                preferred_element_type=f32) + vb_ref[0]
    h = v * (g * jax.nn.sigmoid(g))
    oe_s[:] += jnp.dot(h.astype(bf16), ww_ref[0].astype(bf16),
                       preferred_element_type=f32)

    @pl.when(f == NF - 1)
    def _combine():
        contrib = oe_s[:] * w_s[:]
        out_ref[:] += jax.lax.dot_general(
            onehot_s[:], contrib.astype(bf16),
            dimension_numbers=(((0,), (0,)), ((), ())),
            preferred_element_type=f32)


@functools.partial(jax.jit, static_argnames=("interpret",))
def _moe(logits, xf, gate_W, gate_b, val_W, val_b, wo_W, wo_b,
         interpret=False):
    ti, rank, w = pl.pallas_call(
        _dispatch_kernel,
        out_shape=[
            jax.ShapeDtypeStruct((N, K), jnp.int32),
            jax.ShapeDtypeStruct((E, 1, N), jnp.int32),
            jax.ShapeDtypeStruct((E, 1, N), jnp.float32),
        ],
        interpret=interpret,
    )(logits)

    out = pl.pallas_call(
        _ffn_kernel,
        grid=(E, NF),
        in_specs=[
            pl.BlockSpec((1, 1, N), lambda e, f: (e, 0, 0)),
            pl.BlockSpec((1, 1, N), lambda e, f: (e, 0, 0)),
            pl.BlockSpec((N, D), lambda e, f: (0, 0)),
            pl.BlockSpec((1, D, FT), lambda e, f: (e, 0, f)),
            pl.BlockSpec((1, 1, FT), lambda e, f: (e, 0, f)),
            pl.BlockSpec((1, D, FT), lambda e, f: (e, 0, f)),
            pl.BlockSpec((1, 1, FT), lambda e, f: (e, 0, f)),
            pl.BlockSpec((1, FT, D), lambda e, f: (e, f, 0)),
            pl.BlockSpec((1, 1, D), lambda e, f: (e, 0, 0)),
        ],
        out_specs=pl.BlockSpec((N, D), lambda e, f: (0, 0)),
        out_shape=jax.ShapeDtypeStruct((N, D), jnp.float32),
        scratch_shapes=[
            pltpu.VMEM((CAP, N), jnp.bfloat16),
            pltpu.VMEM((CAP, D), jnp.bfloat16),
            pltpu.VMEM((CAP, D), jnp.float32),
            pltpu.VMEM((CAP, 1), jnp.float32),
        ],
        interpret=interpret,
    )(rank, w, xf.astype(jnp.bfloat16), gate_W,
      gate_b.reshape(E, 1, F), val_W,
      val_b.reshape(E, 1, F), wo_W,
      wo_b.reshape(E, 1, D))
    return out, ti


def kernel(x, Wr, br, gate_W, gate_b, val_W, val_b, wo_W, wo_b):
    b, s, d = x.shape
    xf = x.reshape(b * s, d)
    logits = xf @ Wr + br
    out, ti = _moe(logits, xf, gate_W, gate_b, val_W, val_b, wo_W, wo_b)
    return out.reshape(b, s, d), (logits, ti)
